# baseline (speedup 1.0000x reference)
import functools

import jax
import jax.numpy as jnp
from jax import lax
from jax.experimental import pallas as pl
from jax.experimental.pallas import tpu as pltpu
from jax.experimental.pallas import tpu_sc as plsc

L = 16
NW = 32
NB = 8
N = 2048
CHUNK = (NB * N) // NW
WPB = N // CHUNK
PV = CHUNK // L
TV = N // L
UNROLL = 4

_GATHER_DNUMS = lax.GatherDimensionNumbers(
    offset_dims=(), collapsed_slice_dims=(0,), start_index_map=(0,))


def _lane_gather(v, idx):
    return lax.gather(v, idx[:, None], _GATHER_DNUMS, (1,),
                      mode=lax.GatherScatterMode.PROMISE_IN_BOUNDS)


def _bf16_rne(x):
    u = lax.bitcast_convert_type(x, jnp.int32)
    r = (u + jnp.int32(0x7FFF) + ((u >> 16) & 1)) & jnp.int32(-0x10000)
    return lax.bitcast_convert_type(r, jnp.float32)


def _nn_body(pred_hbm, target_hbm, out_hbm,
             px, py, pz, tx, ty, tz,
             pxb, pyb, pzb, ra, txb, tyb, tzb, rb, ob):
    w = lax.axis_index("s") * 2 + lax.axis_index("c")
    b = w // WPB
    off = (w % WPB) * CHUNK

    pltpu.sync_copy(pred_hbm.at[pl.ds((b * 3 + 0) * N + off, CHUNK)], px)
    pltpu.sync_copy(pred_hbm.at[pl.ds((b * 3 + 1) * N + off, CHUNK)], py)
    pltpu.sync_copy(pred_hbm.at[pl.ds((b * 3 + 2) * N + off, CHUNK)], pz)
    pltpu.sync_copy(target_hbm.at[pl.ds((b * 3 + 0) * N, N)], tx)
    pltpu.sync_copy(target_hbm.at[pl.ds((b * 3 + 1) * N, N)], ty)
    pltpu.sync_copy(target_hbm.at[pl.ds((b * 3 + 2) * N, N)], tz)

    lanes = lax.iota(jnp.int32, L)
    inf = jnp.full((L,), jnp.inf, dtype=jnp.float32)
    two = jnp.float32(2.0)

    def tprep(jt, carry):
        s = pl.ds(jt * L, L)
        xv, yv, zv = tx[s], ty[s], tz[s]
        txb[s] = _bf16_rne(xv) * two
        tyb[s] = _bf16_rne(yv) * two
        tzb[s] = _bf16_rne(zv) * two
        rb[s] = (xv * xv + yv * yv) + zv * zv
        return carry

    lax.fori_loop(0, TV, tprep, 0, unroll=1)

    def pprep(ip, carry):
        s = pl.ds(ip * L, L)
        xv, yv, zv = px[s], py[s], pz[s]
        pxb[s] = _bf16_rne(xv)
        pyb[s] = _bf16_rne(yv)
        pzb[s] = _bf16_rne(zv)
        ra[s] = (xv * xv + yv * yv) + zv * zv
        return carry

    lax.fori_loop(0, PV, pprep, 0, unroll=1)

    def pred_loop(ip, carry):
        s = pl.ds(ip * L, L)
        pxv, pyv, pzv = pxb[s], pyb[s], pzb[s]
        rav = ra[s]

        def rot_loop(r, best):
            bdg, bjg = best
            idx_f = jnp.bitwise_and(lanes + r, L - 1)
            idx_b = jnp.bitwise_and(lanes - r, L - 1)
            pxr = _lane_gather(pxv, idx_f)
            pyr = _lane_gather(pyv, idx_f)
            pzr = _lane_gather(pzv, idx_f)
            rar = _lane_gather(rav, idx_f)

            def tgt_loop(jt, acc):
                bd, bj = acc
                base = jt * (UNROLL * L)
                for k in range(UNROLL):
                    o = base + k * L
                    so = pl.ds(o, L)
                    m2 = (pxr * txb[so] + pzr * tzb[so]) + pyr * tyb[so]
                    d = (rar - m2) + rb[so]
                    jv = lanes + o
                    take = d < bd
                    bd = jnp.where(take, d, bd)
                    bj = jnp.where(take, jv, bj)
                return bd, bj

            bd, bj = lax.fori_loop(0, TV // UNROLL, tgt_loop,
                                   (inf, jnp.zeros((L,), jnp.int32)),
                                   unroll=1)
            bdr = _lane_gather(bd, idx_b)
            bjr = _lane_gather(bj, idx_b)
            take = (bdr < bdg) | ((bdr == bdg) & (bjr < bjg))
            return (jnp.where(take, bdr, bdg), jnp.where(take, bjr, bjg))

        _, bj = lax.fori_loop(0, L, rot_loop,
                              (inf, jnp.zeros((L,), jnp.int32)),
                              unroll=1)

        gx = plsc.load_gather(tx, [bj])
        gy = plsc.load_gather(ty, [bj])
        gz = plsc.load_gather(tz, [bj])
        dx = px[s] - gx
        dy = py[s] - gy
        dz = pz[s] - gz
        d2 = (dx * dx + dy * dy) + dz * dz

        yi = lax.bitcast_convert_type(d2, jnp.int32)
        y = lax.bitcast_convert_type(
            (yi >> 1) + jnp.int32(0x1FBD1DF5), jnp.float32)
        y = 0.5 * (y + d2 / y)
        y = 0.5 * (y + d2 / y)
        y = 0.5 * (y + d2 / y)
        ob[s] = y
        return carry

    lax.fori_loop(0, PV, pred_loop, 0, unroll=1)
    pltpu.sync_copy(ob, out_hbm.at[pl.ds(w * CHUNK, CHUNK)])


@jax.jit
def kernel(pred, target):
    B, T, d, n = pred.shape
    pred_flat = pred.reshape(B * T * d * n)
    target_flat = target.reshape(B * T * d * n)
    nn = pl.kernel(
        _nn_body,
        out_type=jax.ShapeDtypeStruct((NB * N,), jnp.float32),
        mesh=plsc.VectorSubcoreMesh(core_axis_name="c", subcore_axis_name="s"),
        compiler_params=pltpu.CompilerParams(needs_layout_passes=False),
        scratch_types=[
            pltpu.VMEM((CHUNK,), jnp.float32),
            pltpu.VMEM((CHUNK,), jnp.float32),
            pltpu.VMEM((CHUNK,), jnp.float32),
            pltpu.VMEM((N,), jnp.float32),
            pltpu.VMEM((N,), jnp.float32),
            pltpu.VMEM((N,), jnp.float32),
            pltpu.VMEM((CHUNK,), jnp.float32),
            pltpu.VMEM((CHUNK,), jnp.float32),
            pltpu.VMEM((CHUNK,), jnp.float32),
            pltpu.VMEM((CHUNK,), jnp.float32),
            pltpu.VMEM((N,), jnp.float32),
            pltpu.VMEM((N,), jnp.float32),
            pltpu.VMEM((N,), jnp.float32),
            pltpu.VMEM((N,), jnp.float32),
            pltpu.VMEM((CHUNK,), jnp.float32),
        ],
    )
    out = nn(pred_flat, target_flat)
    return out.reshape(NB, N)


# speedup vs baseline: 1.7547x; 1.7547x over previous
import functools

import jax
import jax.numpy as jnp
from jax import lax
from jax.experimental import pallas as pl
from jax.experimental.pallas import tpu as pltpu
from jax.experimental.pallas import tpu_sc as plsc

L = 16
NW = 32
NB = 8
N = 2048
CHUNK = (NB * N) // NW
WPB = N // CHUNK
PV = CHUNK // L
PC = 512
JC = 512


def _argmin_body(predT_ref, pred_ref, tgt_ref, idx_ref):
    aT = predT_ref[0]
    ap = pred_ref[0]
    t = tgt_ref[0]
    ra = (ap[0] * ap[0] + ap[1] * ap[1]) + ap[2] * ap[2]
    bd = jnp.full((PC,), jnp.inf, jnp.float32)
    bj = jnp.zeros((PC,), jnp.int32)
    for c in range(N // JC):
        tj = t[:, c * JC:(c + 1) * JC]
        rb = (tj[0] * tj[0] + tj[1] * tj[1]) + tj[2] * tj[2]
        m = lax.dot_general(aT, tj, (((1,), (0,)), ((), ())),
                            preferred_element_type=jnp.float32)
        d = (ra[:, None] - 2.0 * m) + rb[None, :]
        jj = lax.broadcasted_iota(jnp.int32, (PC, JC), 1)
        rowmin = jnp.min(d, axis=1)
        sel = jnp.where(d == rowmin[:, None], jj, jnp.int32(N))
        rowidx = jnp.min(sel, axis=1) + c * JC
        take = (rowmin < bd) | ((rowmin == bd) & (rowidx < bj))
        bd = jnp.where(take, rowmin, bd)
        bj = jnp.where(take, rowidx, bj)
    idx_ref[0, 0] = bj


_tc_argmin = pl.pallas_call(
    _argmin_body,
    grid=(NB, N // PC),
    in_specs=[
        pl.BlockSpec((1, PC, 3), lambda b, i: (b, i, 0)),
        pl.BlockSpec((1, 3, PC), lambda b, i: (b, 0, i)),
        pl.BlockSpec((1, 3, N), lambda b, i: (b, 0, 0)),
    ],
    out_specs=pl.BlockSpec((1, 1, PC), lambda b, i: (b * (N // PC) + i, 0, 0)),
    out_shape=jax.ShapeDtypeStruct((NB * (N // PC), 1, PC), jnp.int32),
)


def _gather_body(pred_hbm, target_hbm, idx_hbm, out_hbm,
                 px, py, pz, tx, ty, tz, ib, ob):
    w = lax.axis_index("s") * 2 + lax.axis_index("c")
    b = w // WPB
    off = (w % WPB) * CHUNK

    pltpu.sync_copy(pred_hbm.at[pl.ds((b * 3 + 0) * N + off, CHUNK)], px)
    pltpu.sync_copy(pred_hbm.at[pl.ds((b * 3 + 1) * N + off, CHUNK)], py)
    pltpu.sync_copy(pred_hbm.at[pl.ds((b * 3 + 2) * N + off, CHUNK)], pz)
    pltpu.sync_copy(target_hbm.at[pl.ds((b * 3 + 0) * N, N)], tx)
    pltpu.sync_copy(target_hbm.at[pl.ds((b * 3 + 1) * N, N)], ty)
    pltpu.sync_copy(target_hbm.at[pl.ds((b * 3 + 2) * N, N)], tz)
    pltpu.sync_copy(idx_hbm.at[pl.ds(w * CHUNK, CHUNK)], ib)

    def pred_loop(ip, carry):
        s = pl.ds(ip * L, L)
        bj = ib[s]
        gx = plsc.load_gather(tx, [bj])
        gy = plsc.load_gather(ty, [bj])
        gz = plsc.load_gather(tz, [bj])
        dx = px[s] - gx
        dy = py[s] - gy
        dz = pz[s] - gz
        d2 = (dx * dx + dy * dy) + dz * dz

        yi = lax.bitcast_convert_type(d2, jnp.int32)
        y = lax.bitcast_convert_type(
            (yi >> 1) + jnp.int32(0x1FBD1DF5), jnp.float32)
        y = 0.5 * (y + d2 / y)
        y = 0.5 * (y + d2 / y)
        y = 0.5 * (y + d2 / y)
        ob[s] = y
        return carry

    lax.fori_loop(0, PV, pred_loop, 0, unroll=1)
    pltpu.sync_copy(ob, out_hbm.at[pl.ds(w * CHUNK, CHUNK)])


_sc_gather = pl.kernel(
    _gather_body,
    out_type=jax.ShapeDtypeStruct((NB * N,), jnp.float32),
    mesh=plsc.VectorSubcoreMesh(core_axis_name="c", subcore_axis_name="s"),
    compiler_params=pltpu.CompilerParams(needs_layout_passes=False),
    scratch_types=[
        pltpu.VMEM((CHUNK,), jnp.float32),
        pltpu.VMEM((CHUNK,), jnp.float32),
        pltpu.VMEM((CHUNK,), jnp.float32),
        pltpu.VMEM((N,), jnp.float32),
        pltpu.VMEM((N,), jnp.float32),
        pltpu.VMEM((N,), jnp.float32),
        pltpu.VMEM((CHUNK,), jnp.int32),
        pltpu.VMEM((CHUNK,), jnp.float32),
    ],
)


@jax.jit
def kernel(pred, target):
    B, T, d, n = pred.shape
    pred2 = pred.reshape(NB, 3, N)
    target2 = target.reshape(NB, 3, N)
    predT = jnp.transpose(pred2, (0, 2, 1))
    idx = _tc_argmin(predT, pred2, target2).reshape(NB * N)
    out = _sc_gather(pred2.reshape(-1), target2.reshape(-1), idx)
    return out.reshape(NB, N)
